# baseline (speedup 1.0000x reference)
import functools

import jax
import jax.numpy as jnp
from jax.experimental import pallas as pl
from jax.experimental.pallas import tpu as pltpu
from jax.experimental.pallas import tpu_sc as plsc

D_MODEL = 512
NUM_TILES = 16
K_SEL = 4
D_FF = 1024
BATCH = 4096
NUM_OPS = 8
OP_EMB = 32
BLK = 256
NB = BATCH // BLK

A_N = BATCH * K_SEL
BM = 128
PADN = A_N + NUM_TILES * BM
G = PADN // BM

NC = 2
NS = 16
NW = NC * NS
TPW = BATCH // NW
LANES = 16


def _dot(x, y):
    return jax.lax.dot_general(x, y, (((1,), (0,)), ((), ())),
                               preferred_element_type=jnp.float32)


def _dot_t(x, y):
    return jax.lax.dot_general(x, y, (((1,), (1,)), ((), ())),
                               preferred_element_type=jnp.float32)


def _prologue_body(op_ref, a_ref, b_ref, c_ref, op_table_ref, wp_ref,
                   bp_ref, wm_ref, bm_ref, sig_ref, tri_ref,
                   x_ref, topi_ref, gates_ref, aux_ref, rank_ref,
                   offs_ref, bexp_ref,
                   acc_imp, cnt_s):
    i = pl.program_id(0)
    f32 = jnp.float32
    i32 = jnp.int32

    iota8 = jax.lax.broadcasted_iota(i32, (BLK, 8), 1)
    op1h = (op_ref[:, :] == iota8).astype(f32)
    abits = ((a_ref[:, :] >> iota8) & 1).astype(f32)
    bbits = ((b_ref[:, :] >> iota8) & 1).astype(f32)
    c_f = c_ref[:, :].astype(f32)

    op_emb = _dot(op1h, op_table_ref[:, :])
    feats = jnp.concatenate(
        [op_emb, abits, bbits, c_f, jnp.zeros((BLK, 15), f32)], axis=1)
    x = _dot(feats, wp_ref[:, :]) + bp_ref[:, :]
    x = x + (_dot(jnp.tanh(x), wm_ref[:, :]) + bm_ref[:, :])
    x_ref[:, :] = x

    sig = jnp.tanh(sig_ref[:, :])
    logits = _dot_t(x, sig)

    iota_t = jax.lax.broadcasted_iota(i32, (BLK, NUM_TILES), 1)
    rem = logits
    topi_cols = []
    topv_cols = []
    for _ in range(K_SEL):
        m = jnp.max(rem, axis=1, keepdims=True)
        ismax = rem == m
        idx = jnp.min(jnp.where(ismax, iota_t, NUM_TILES + 1), axis=1,
                      keepdims=True)
        topi_cols.append(idx)
        topv_cols.append(m)
        rem = jnp.where(iota_t == idx, f32(-1e30), rem)

    topv = jnp.concatenate(topv_cols, axis=1)
    e_top = jnp.exp(topv - topv_cols[0])
    gates = e_top / jnp.sum(e_top, axis=1, keepdims=True)
    topi_ref[:, :] = jnp.concatenate(topi_cols, axis=1)
    gates_ref[:, :] = gates

    oh_k = [(iota_t == topi_cols[k]).astype(f32) for k in range(K_SEL)]
    ind = oh_k[0] + oh_k[1] + oh_k[2] + oh_k[3]

    cnt_prev = jnp.where(i > 0, cnt_s[:, :], 0.0)
    within = _dot(tri_ref[:, :], ind) + cnt_prev
    rank_cols = [jnp.sum(within * oh_k[k], axis=1, keepdims=True)
                 for k in range(K_SEL)]
    rank_ref[:, :] = jnp.concatenate(rank_cols, axis=1).astype(i32)
    cnt_new = cnt_prev + jnp.sum(ind, axis=0, keepdims=True)
    cnt_s[:, :] = cnt_new

    mfull = jnp.max(logits, axis=1, keepdims=True)
    p = jnp.exp(logits - mfull)
    probs = p / jnp.sum(p, axis=1, keepdims=True)

    @pl.when(i == 0)
    def _():
        acc_imp[:, :] = probs

    @pl.when(i > 0)
    def _():
        acc_imp[:, :] = acc_imp[:, :] + probs

    @pl.when(i == NB - 1)
    def _():
        cnti = cnt_new.astype(i32)
        pc = ((cnti + BM - 1) & ~(BM - 1)).astype(f32)
        r16 = jax.lax.broadcasted_iota(i32, (NUM_TILES, NUM_TILES), 0)
        c16 = jax.lax.broadcasted_iota(i32, (NUM_TILES, NUM_TILES), 1)
        strict = (r16 < c16).astype(f32)
        offs = jax.lax.dot_general(pc, strict, (((1,), (0,)), ((), ())),
                                   preferred_element_type=f32,
                                   precision=jax.lax.Precision.HIGHEST)
        offs_ref[:, :] = offs.astype(i32)
        g_r = (jax.lax.broadcasted_iota(i32, (G, NUM_TILES), 0)
               .astype(f32) * BM)
        ge = (g_r >= offs).astype(i32)
        bexp_ref[:, :] = jnp.sum(ge, axis=1, keepdims=True) - 1

        s_imp = jnp.sum(acc_imp[:, :], axis=0, keepdims=True)
        sparsity = NUM_TILES * jnp.sum(s_imp * cnt_new) / (BATCH * BATCH)
        abs_sig = jnp.abs(sig)
        ternary = jnp.mean(abs_sig * (1.0 - abs_sig))
        norm = jnp.sqrt(jnp.sum(sig * sig, axis=1, keepdims=True))
        sig_n = sig / (norm + 1e-6)
        cos = _dot_t(sig_n, sig_n)
        trace = jnp.sum(jnp.where(r16 == c16, cos, 0.0))
        diversity = (jnp.sum(cos) - trace) / (NUM_TILES * (NUM_TILES - 1))
        aux = 0.01 * ternary + 0.005 * sparsity + 0.01 * diversity
        aux_ref[:, :] = jnp.reshape(aux, (1, 1))


def _ffn_body(bexp_ref, xs_ref, w1_ref, b1_ref, w2_ref, b2_ref, ys_ref):
    h = jnp.maximum(_dot(xs_ref[:, :], w1_ref[0]) + b1_ref[0], 0.0)
    ys_ref[:, :] = _dot(h, w2_ref[0]) + b2_ref[0]


def _sc_pos(ti_v, rk_v, offs_v, idx2):
    offs_reg = offs_v[:]
    for k in range(K_SEL):
        for j in range(TPW // LANES):
            sl = pl.ds(j * LANES, LANES)
            tv = ti_v[k, sl]
            rv = rk_v[k, sl]
            ov = jax.lax.gather(
                offs_reg, tv[:, None],
                jax.lax.GatherDimensionNumbers(
                    offset_dims=(), collapsed_slice_dims=(0,),
                    start_index_map=(0,)),
                (1,), mode=jax.lax.GatherScatterMode.PROMISE_IN_BOUNDS)
            idx2[k, sl] = ov + rv


def _sc_scatter_body(x_hbm, tiw_hbm, rkw_hbm, offs_hbm, xs_hbm,
                     xv, ti_v, rk_v, offs_v, idx2, sem):
    wid = jax.lax.axis_index("s") * NC + jax.lax.axis_index("c")
    tbase = wid * TPW
    pltpu.sync_copy(x_hbm.at[pl.ds(tbase, TPW)], xv)
    pltpu.sync_copy(tiw_hbm.at[wid], ti_v)
    pltpu.sync_copy(rkw_hbm.at[wid], rk_v)
    pltpu.sync_copy(offs_hbm, offs_v)
    _sc_pos(ti_v, rk_v, offs_v, idx2)
    descs = [pltpu.async_copy(xv, xs_hbm.at[idx2.at[k]], sem)
             for k in range(K_SEL)]
    for dsc in descs:
        dsc.wait()


def _sc_gather_body(ys_hbm, tiw_hbm, rkw_hbm, offs_hbm, m4_hbm,
                    rows, ti_v, rk_v, offs_v, idx2, sem):
    wid = jax.lax.axis_index("s") * NC + jax.lax.axis_index("c")
    tbase = wid * TPW
    pltpu.sync_copy(tiw_hbm.at[wid], ti_v)
    pltpu.sync_copy(rkw_hbm.at[wid], rk_v)
    pltpu.sync_copy(offs_hbm, offs_v)
    _sc_pos(ti_v, rk_v, offs_v, idx2)
    for k in range(K_SEL):
        pltpu.async_copy(ys_hbm.at[idx2.at[k]], rows, sem).wait()
        pltpu.sync_copy(rows, m4_hbm.at[k, pl.ds(tbase, TPW)])


def _head_body(x_ref, m4_ref, g_ref, wh1_ref, bh1_ref, wh2_ref, bh2_ref,
               out_ref):
    f32 = jnp.float32
    moe = jnp.zeros((BLK, D_MODEL), f32)
    for k in range(K_SEL):
        moe = moe + g_ref[:, k:k + 1] * m4_ref[k]
    out = x_ref[:, :] + moe
    hh = jnp.maximum(_dot(out, wh1_ref[:, :]) + bh1_ref[:, :], 0.0)
    out_ref[:, :] = jax.nn.sigmoid(_dot(hh, wh2_ref[:, :]) + bh2_ref[:, :])


def kernel(op_idx, a, b, c, op_table, Wp, bp, Wm, bm, sig_raw,
           W1, b1, W2, b2, Wh1, bh1, Wh2, bh2):
    f32 = jnp.float32
    i32 = jnp.int32
    op_col = op_idx.reshape(BATCH, 1)
    a_col = a.reshape(BATCH, 1)
    b_col = b.reshape(BATCH, 1)
    c_col = c.reshape(BATCH, 1)
    wp64 = jnp.zeros((64, D_MODEL), f32).at[:49].set(Wp)
    bp2 = bp.reshape(1, D_MODEL)
    bm2 = bm.reshape(1, D_MODEL)
    r_b = jax.lax.broadcasted_iota(i32, (BLK, BLK), 0)
    c_b = jax.lax.broadcasted_iota(i32, (BLK, BLK), 1)
    tri = (c_b < r_b).astype(f32)

    col_spec = pl.BlockSpec((BLK, 1), lambda i: (i, 0))
    full = pl.BlockSpec(index_map=lambda i: (0, 0))

    x, topi, gates, aux, rank, offs16, bexp = pl.pallas_call(
        _prologue_body,
        grid=(NB,),
        in_specs=[col_spec, col_spec, col_spec, col_spec,
                  full, full, full, full, full, full, full],
        out_specs=[
            pl.BlockSpec((BLK, D_MODEL), lambda i: (i, 0)),
            pl.BlockSpec((BLK, K_SEL), lambda i: (i, 0)),
            pl.BlockSpec((BLK, K_SEL), lambda i: (i, 0)),
            pl.BlockSpec((1, 1), lambda i: (0, 0)),
            pl.BlockSpec((BLK, K_SEL), lambda i: (i, 0)),
            pl.BlockSpec((1, NUM_TILES), lambda i: (0, 0)),
            pl.BlockSpec((G, 1), lambda i: (0, 0)),
        ],
        out_shape=[
            jax.ShapeDtypeStruct((BATCH, D_MODEL), f32),
            jax.ShapeDtypeStruct((BATCH, K_SEL), i32),
            jax.ShapeDtypeStruct((BATCH, K_SEL), f32),
            jax.ShapeDtypeStruct((1, 1), f32),
            jax.ShapeDtypeStruct((BATCH, K_SEL), i32),
            jax.ShapeDtypeStruct((1, NUM_TILES), i32),
            jax.ShapeDtypeStruct((G, 1), i32),
        ],
        scratch_shapes=[pltpu.VMEM((BLK, NUM_TILES), f32),
                        pltpu.VMEM((1, NUM_TILES), f32)],
    )(op_col, a_col, b_col, c_col, op_table, wp64, bp2, Wm, bm2, sig_raw,
      tri)

    tiw = topi.reshape(NW, TPW, K_SEL).transpose(0, 2, 1)
    rkw = rank.reshape(NW, TPW, K_SEL).transpose(0, 2, 1)
    offs_flat = offs16.reshape(NUM_TILES)

    mesh = plsc.VectorSubcoreMesh(core_axis_name="c", subcore_axis_name="s")
    sc_scr = [pltpu.VMEM((K_SEL, TPW), i32),
              pltpu.VMEM((K_SEL, TPW), i32),
              pltpu.VMEM((NUM_TILES,), i32),
              pltpu.VMEM((K_SEL, TPW), i32),
              pltpu.SemaphoreType.DMA]

    xs = pl.kernel(
        _sc_scatter_body, mesh=mesh,
        out_type=jax.ShapeDtypeStruct((PADN, D_MODEL), f32),
        scratch_types=[pltpu.VMEM((TPW, D_MODEL), f32)] + sc_scr,
    )(x, tiw, rkw, offs_flat)

    ys = pl.pallas_call(
        _ffn_body,
        grid_spec=pltpu.PrefetchScalarGridSpec(
            num_scalar_prefetch=1,
            grid=(G,),
            in_specs=[
                pl.BlockSpec((BM, D_MODEL), lambda g, be: (g, 0)),
                pl.BlockSpec((1, D_MODEL, D_FF), lambda g, be: (be[g], 0, 0)),
                pl.BlockSpec((1, 1, D_FF), lambda g, be: (be[g], 0, 0)),
                pl.BlockSpec((1, D_FF, D_MODEL), lambda g, be: (be[g], 0, 0)),
                pl.BlockSpec((1, 1, D_MODEL), lambda g, be: (be[g], 0, 0)),
            ],
            out_specs=pl.BlockSpec((BM, D_MODEL), lambda g, be: (g, 0)),
        ),
        out_shape=jax.ShapeDtypeStruct((PADN, D_MODEL), f32),
    )(bexp.reshape(G), xs, W1, b1.reshape(NUM_TILES, 1, D_FF), W2,
      b2.reshape(NUM_TILES, 1, D_MODEL))

    m4 = pl.kernel(
        _sc_gather_body, mesh=mesh,
        out_type=jax.ShapeDtypeStruct((K_SEL, BATCH, D_MODEL), f32),
        scratch_types=[pltpu.VMEM((TPW, D_MODEL), f32)] + sc_scr,
    )(ys, tiw, rkw, offs_flat)

    wh2p = jnp.zeros((64, 128), f32).at[:, :8].set(Wh2)
    bh2p = jnp.zeros((1, 128), f32).at[0, :8].set(bh2)
    bh1p = bh1.reshape(1, 64)

    res128 = pl.pallas_call(
        _head_body,
        grid=(NB,),
        in_specs=[
            pl.BlockSpec((BLK, D_MODEL), lambda i: (i, 0)),
            pl.BlockSpec((K_SEL, BLK, D_MODEL), lambda i: (0, i, 0)),
            pl.BlockSpec((BLK, K_SEL), lambda i: (i, 0)),
            full, full, full, full,
        ],
        out_specs=pl.BlockSpec((BLK, 128), lambda i: (i, 0)),
        out_shape=jax.ShapeDtypeStruct((BATCH, 128), f32),
    )(x, m4, gates, Wh1, bh1p, wh2p, bh2p)

    result = res128[:, :8]
    return result, topi, aux.reshape(())


# speedup vs baseline: 1.1953x; 1.1953x over previous
import functools

import jax
import jax.numpy as jnp
from jax.experimental import pallas as pl
from jax.experimental.pallas import tpu as pltpu
from jax.experimental.pallas import tpu_sc as plsc

D_MODEL = 512
NUM_TILES = 16
K_SEL = 4
D_FF = 1024
BATCH = 4096
NUM_OPS = 8
OP_EMB = 32
BLK = 256
NB = BATCH // BLK

A_N = BATCH * K_SEL
BM = 512
PADN = A_N + NUM_TILES * BM
G = PADN // BM

NC = 2
NS = 16
NW = NC * NS
TPW = BATCH // NW
LANES = 16


def _dot(x, y):
    return jax.lax.dot_general(x, y, (((1,), (0,)), ((), ())),
                               preferred_element_type=jnp.float32)


def _dot_t(x, y):
    return jax.lax.dot_general(x, y, (((1,), (1,)), ((), ())),
                               preferred_element_type=jnp.float32)


def _prologue_body(op_ref, a_ref, b_ref, c_ref, op_table_ref, wp_ref,
                   bp_ref, wm_ref, bm_ref, sig_ref, tri_ref,
                   x_ref, topi_ref, gates_ref, aux_ref, rank_ref,
                   offs_ref, bexp_ref,
                   acc_imp, cnt_s):
    i = pl.program_id(0)
    f32 = jnp.float32
    i32 = jnp.int32

    iota8 = jax.lax.broadcasted_iota(i32, (BLK, 8), 1)
    op1h = (op_ref[:, :] == iota8).astype(f32)
    abits = ((a_ref[:, :] >> iota8) & 1).astype(f32)
    bbits = ((b_ref[:, :] >> iota8) & 1).astype(f32)
    c_f = c_ref[:, :].astype(f32)

    op_emb = _dot(op1h, op_table_ref[:, :])
    feats = jnp.concatenate(
        [op_emb, abits, bbits, c_f, jnp.zeros((BLK, 15), f32)], axis=1)
    x = _dot(feats, wp_ref[:, :]) + bp_ref[:, :]
    x = x + (_dot(jnp.tanh(x), wm_ref[:, :]) + bm_ref[:, :])
    x_ref[:, :] = x

    sig = jnp.tanh(sig_ref[:, :])
    logits = _dot_t(x, sig)

    iota_t = jax.lax.broadcasted_iota(i32, (BLK, NUM_TILES), 1)
    rem = logits
    topi_cols = []
    topv_cols = []
    for _ in range(K_SEL):
        m = jnp.max(rem, axis=1, keepdims=True)
        ismax = rem == m
        idx = jnp.min(jnp.where(ismax, iota_t, NUM_TILES + 1), axis=1,
                      keepdims=True)
        topi_cols.append(idx)
        topv_cols.append(m)
        rem = jnp.where(iota_t == idx, f32(-1e30), rem)

    topv = jnp.concatenate(topv_cols, axis=1)
    e_top = jnp.exp(topv - topv_cols[0])
    gates = e_top / jnp.sum(e_top, axis=1, keepdims=True)
    topi_ref[:, :] = jnp.concatenate(topi_cols, axis=1)
    gates_ref[:, :] = gates

    oh_k = [(iota_t == topi_cols[k]).astype(f32) for k in range(K_SEL)]
    ind = oh_k[0] + oh_k[1] + oh_k[2] + oh_k[3]

    cnt_prev = jnp.where(i > 0, cnt_s[:, :], 0.0)
    within = _dot(tri_ref[:, :], ind) + cnt_prev
    rank_cols = [jnp.sum(within * oh_k[k], axis=1, keepdims=True)
                 for k in range(K_SEL)]
    rank_ref[:, :] = jnp.concatenate(rank_cols, axis=1).astype(i32)
    cnt_new = cnt_prev + jnp.sum(ind, axis=0, keepdims=True)
    cnt_s[:, :] = cnt_new

    mfull = jnp.max(logits, axis=1, keepdims=True)
    p = jnp.exp(logits - mfull)
    probs = p / jnp.sum(p, axis=1, keepdims=True)

    @pl.when(i == 0)
    def _():
        acc_imp[:, :] = probs

    @pl.when(i > 0)
    def _():
        acc_imp[:, :] = acc_imp[:, :] + probs

    @pl.when(i == NB - 1)
    def _():
        cnti = cnt_new.astype(i32)
        pc = ((cnti + BM - 1) & ~(BM - 1)).astype(f32)
        r16 = jax.lax.broadcasted_iota(i32, (NUM_TILES, NUM_TILES), 0)
        c16 = jax.lax.broadcasted_iota(i32, (NUM_TILES, NUM_TILES), 1)
        strict = (r16 < c16).astype(f32)
        offs = jax.lax.dot_general(pc, strict, (((1,), (0,)), ((), ())),
                                   preferred_element_type=f32,
                                   precision=jax.lax.Precision.HIGHEST)
        offs_ref[:, :] = offs.astype(i32)
        g_r = (jax.lax.broadcasted_iota(i32, (G, NUM_TILES), 0)
               .astype(f32) * BM)
        ge = (g_r >= offs).astype(i32)
        bexp_ref[:, :] = jnp.sum(ge, axis=1, keepdims=True) - 1

        s_imp = jnp.sum(acc_imp[:, :], axis=0, keepdims=True)
        sparsity = NUM_TILES * jnp.sum(s_imp * cnt_new) / (BATCH * BATCH)
        abs_sig = jnp.abs(sig)
        ternary = jnp.mean(abs_sig * (1.0 - abs_sig))
        norm = jnp.sqrt(jnp.sum(sig * sig, axis=1, keepdims=True))
        sig_n = sig / (norm + 1e-6)
        cos = _dot_t(sig_n, sig_n)
        trace = jnp.sum(jnp.where(r16 == c16, cos, 0.0))
        diversity = (jnp.sum(cos) - trace) / (NUM_TILES * (NUM_TILES - 1))
        aux = 0.01 * ternary + 0.005 * sparsity + 0.01 * diversity
        aux_ref[:, :] = jnp.reshape(aux, (1, 1))


def _ffn_body(bexp_ref, xs_ref, w1_ref, b1_ref, w2_ref, b2_ref, ys_ref):
    h = jnp.maximum(_dot(xs_ref[:, :], w1_ref[0]) + b1_ref[0], 0.0)
    ys_ref[:, :] = _dot(h, w2_ref[0]) + b2_ref[0]


def _sc_pos(ti_v, rk_v, offs_v, idx2):
    offs_reg = offs_v[:]
    for k in range(K_SEL):
        for j in range(TPW // LANES):
            sl = pl.ds(j * LANES, LANES)
            tv = ti_v[k, sl]
            rv = rk_v[k, sl]
            ov = jax.lax.gather(
                offs_reg, tv[:, None],
                jax.lax.GatherDimensionNumbers(
                    offset_dims=(), collapsed_slice_dims=(0,),
                    start_index_map=(0,)),
                (1,), mode=jax.lax.GatherScatterMode.PROMISE_IN_BOUNDS)
            idx2[k, sl] = ov + rv


def _sc_scatter_body(x_hbm, tiw_hbm, rkw_hbm, offs_hbm, xs_hbm,
                     xv, ti_v, rk_v, offs_v, idx2, sem):
    wid = jax.lax.axis_index("s") * NC + jax.lax.axis_index("c")
    tbase = wid * TPW
    pltpu.sync_copy(x_hbm.at[pl.ds(tbase, TPW)], xv)
    pltpu.sync_copy(tiw_hbm.at[wid], ti_v)
    pltpu.sync_copy(rkw_hbm.at[wid], rk_v)
    pltpu.sync_copy(offs_hbm, offs_v)
    _sc_pos(ti_v, rk_v, offs_v, idx2)
    descs = [pltpu.async_copy(xv, xs_hbm.at[idx2.at[k]], sem)
             for k in range(K_SEL)]
    for dsc in descs:
        dsc.wait()


def _sc_gather_body(ys_hbm, tiw_hbm, rkw_hbm, offs_hbm, m4_hbm,
                    rows, ti_v, rk_v, offs_v, idx2, sem):
    wid = jax.lax.axis_index("s") * NC + jax.lax.axis_index("c")
    tbase = wid * TPW
    pltpu.sync_copy(tiw_hbm.at[wid], ti_v)
    pltpu.sync_copy(rkw_hbm.at[wid], rk_v)
    pltpu.sync_copy(offs_hbm, offs_v)
    _sc_pos(ti_v, rk_v, offs_v, idx2)
    for k in range(K_SEL):
        pltpu.async_copy(ys_hbm.at[idx2.at[k]], rows, sem).wait()
        pltpu.sync_copy(rows, m4_hbm.at[k, pl.ds(tbase, TPW)])


def _head_body(x_ref, m4_ref, g_ref, wh1_ref, bh1_ref, wh2_ref, bh2_ref,
               out_ref):
    f32 = jnp.float32
    moe = jnp.zeros((BLK, D_MODEL), f32)
    for k in range(K_SEL):
        moe = moe + g_ref[:, k:k + 1] * m4_ref[k]
    out = x_ref[:, :] + moe
    hh = jnp.maximum(_dot(out, wh1_ref[:, :]) + bh1_ref[:, :], 0.0)
    out_ref[:, :] = jax.nn.sigmoid(_dot(hh, wh2_ref[:, :]) + bh2_ref[:, :])


def kernel(op_idx, a, b, c, op_table, Wp, bp, Wm, bm, sig_raw,
           W1, b1, W2, b2, Wh1, bh1, Wh2, bh2):
    f32 = jnp.float32
    i32 = jnp.int32
    op_col = op_idx.reshape(BATCH, 1)
    a_col = a.reshape(BATCH, 1)
    b_col = b.reshape(BATCH, 1)
    c_col = c.reshape(BATCH, 1)
    wp64 = jnp.zeros((64, D_MODEL), f32).at[:49].set(Wp)
    bp2 = bp.reshape(1, D_MODEL)
    bm2 = bm.reshape(1, D_MODEL)
    r_b = jax.lax.broadcasted_iota(i32, (BLK, BLK), 0)
    c_b = jax.lax.broadcasted_iota(i32, (BLK, BLK), 1)
    tri = (c_b < r_b).astype(f32)

    col_spec = pl.BlockSpec((BLK, 1), lambda i: (i, 0))
    full = pl.BlockSpec(index_map=lambda i: (0, 0))

    x, topi, gates, aux, rank, offs16, bexp = pl.pallas_call(
        _prologue_body,
        grid=(NB,),
        in_specs=[col_spec, col_spec, col_spec, col_spec,
                  full, full, full, full, full, full, full],
        out_specs=[
            pl.BlockSpec((BLK, D_MODEL), lambda i: (i, 0)),
            pl.BlockSpec((BLK, K_SEL), lambda i: (i, 0)),
            pl.BlockSpec((BLK, K_SEL), lambda i: (i, 0)),
            pl.BlockSpec((1, 1), lambda i: (0, 0)),
            pl.BlockSpec((BLK, K_SEL), lambda i: (i, 0)),
            pl.BlockSpec((1, NUM_TILES), lambda i: (0, 0)),
            pl.BlockSpec((G, 1), lambda i: (0, 0)),
        ],
        out_shape=[
            jax.ShapeDtypeStruct((BATCH, D_MODEL), f32),
            jax.ShapeDtypeStruct((BATCH, K_SEL), i32),
            jax.ShapeDtypeStruct((BATCH, K_SEL), f32),
            jax.ShapeDtypeStruct((1, 1), f32),
            jax.ShapeDtypeStruct((BATCH, K_SEL), i32),
            jax.ShapeDtypeStruct((1, NUM_TILES), i32),
            jax.ShapeDtypeStruct((G, 1), i32),
        ],
        scratch_shapes=[pltpu.VMEM((BLK, NUM_TILES), f32),
                        pltpu.VMEM((1, NUM_TILES), f32)],
    )(op_col, a_col, b_col, c_col, op_table, wp64, bp2, Wm, bm2, sig_raw,
      tri)

    tiw = topi.reshape(NW, TPW, K_SEL).transpose(0, 2, 1)
    rkw = rank.reshape(NW, TPW, K_SEL).transpose(0, 2, 1)
    offs_flat = offs16.reshape(NUM_TILES)

    mesh = plsc.VectorSubcoreMesh(core_axis_name="c", subcore_axis_name="s")
    sc_scr = [pltpu.VMEM((K_SEL, TPW), i32),
              pltpu.VMEM((K_SEL, TPW), i32),
              pltpu.VMEM((NUM_TILES,), i32),
              pltpu.VMEM((K_SEL, TPW), i32),
              pltpu.SemaphoreType.DMA]

    xs = pl.kernel(
        _sc_scatter_body, mesh=mesh,
        out_type=jax.ShapeDtypeStruct((PADN, D_MODEL), f32),
        scratch_types=[pltpu.VMEM((TPW, D_MODEL), f32)] + sc_scr,
    )(x, tiw, rkw, offs_flat)

    ys = pl.pallas_call(
        _ffn_body,
        grid_spec=pltpu.PrefetchScalarGridSpec(
            num_scalar_prefetch=1,
            grid=(G,),
            in_specs=[
                pl.BlockSpec((BM, D_MODEL), lambda g, be: (g, 0)),
                pl.BlockSpec((1, D_MODEL, D_FF), lambda g, be: (be[g], 0, 0)),
                pl.BlockSpec((1, 1, D_FF), lambda g, be: (be[g], 0, 0)),
                pl.BlockSpec((1, D_FF, D_MODEL), lambda g, be: (be[g], 0, 0)),
                pl.BlockSpec((1, 1, D_MODEL), lambda g, be: (be[g], 0, 0)),
            ],
            out_specs=pl.BlockSpec((BM, D_MODEL), lambda g, be: (g, 0)),
        ),
        out_shape=jax.ShapeDtypeStruct((PADN, D_MODEL), f32),
    )(bexp.reshape(G), xs, W1, b1.reshape(NUM_TILES, 1, D_FF), W2,
      b2.reshape(NUM_TILES, 1, D_MODEL))

    m4 = pl.kernel(
        _sc_gather_body, mesh=mesh,
        out_type=jax.ShapeDtypeStruct((K_SEL, BATCH, D_MODEL), f32),
        scratch_types=[pltpu.VMEM((TPW, D_MODEL), f32)] + sc_scr,
    )(ys, tiw, rkw, offs_flat)

    wh2p = jnp.zeros((64, 128), f32).at[:, :8].set(Wh2)
    bh2p = jnp.zeros((1, 128), f32).at[0, :8].set(bh2)
    bh1p = bh1.reshape(1, 64)

    res128 = pl.pallas_call(
        _head_body,
        grid=(NB,),
        in_specs=[
            pl.BlockSpec((BLK, D_MODEL), lambda i: (i, 0)),
            pl.BlockSpec((K_SEL, BLK, D_MODEL), lambda i: (0, i, 0)),
            pl.BlockSpec((BLK, K_SEL), lambda i: (i, 0)),
            full, full, full, full,
        ],
        out_specs=pl.BlockSpec((BLK, 128), lambda i: (i, 0)),
        out_shape=jax.ShapeDtypeStruct((BATCH, 128), f32),
    )(x, m4, gates, Wh1, bh1p, wh2p, bh2p)

    result = res128[:, :8]
    return result, topi, aux.reshape(())
